# baseline (speedup 1.0000x reference)
import functools

import jax
import jax.numpy as jnp
from jax import lax
from jax.experimental import pallas as pl
from jax.experimental.pallas import tpu as pltpu
from jax.experimental.pallas import tpu_sc as plsc

N, E, D_IN, D_E, D_OUT = 10000, 320000, 128, 16, 128

_NC, _NS = 2, 16
_NW = _NC * _NS
_EPW = E // _NW
_CH = 80
_NCH = _EPW // _CH
_RB = 2000
_NRB = _EPW // _RB
_CPB = _RB // _CH
_NPAD = 10240
_RPS = _NPAD // _NS


def _seg_sum_body(
    ef_t_hbm, dst_hbm, out_hbm,
    idx_v, tbuf, rows_v0, rows_v1, acc_sh, lsem, ssem,
):
  c = lax.axis_index("c")
  s = lax.axis_index("s")
  wid = s * _NC + c
  rows = (rows_v0, rows_v1)

  zeros16 = jnp.zeros((16,), jnp.float32)

  def zero_body(i, carry):
    rows_v0[i, :] = zeros16
    return carry

  lax.fori_loop(0, _RPS, zero_body, 0)
  pltpu.sync_copy(
      rows_v0.at[pl.ds(0, _RPS), :], acc_sh.at[pl.ds(s * _RPS, _RPS), :]
  )

  pltpu.sync_copy(dst_hbm.at[1, wid], idx_v)
  plsc.subcore_barrier()

  def start_load(b):
    return pltpu.async_copy(
        ef_t_hbm.at[:, pl.ds(wid * _EPW + b * _RB, _RB)],
        tbuf.at[:, pl.ds(0, _RB)],
        lsem,
    )

  iota16 = lax.iota(jnp.int32, 16)

  def fire_block(b):
    def fire_body(k, carry, b=b):
      pltpu.async_copy(
          rows[b % 2].at[pl.ds(k * _CH, _CH), :],
          acc_sh.at[idx_v.at[b * _CPB + k]],
          ssem,
          add=True,
      )
      return carry

    lax.fori_loop(0, _CPB, fire_body, 0)

  def drain_block(b):
    def drain_body(k, carry, b=b):
      pltpu.make_async_copy(
          rows[b % 2].at[pl.ds(k * _CH, _CH), :],
          acc_sh.at[idx_v.at[b * _CPB + k]],
          ssem,
      ).wait()
      return carry

    lax.fori_loop(0, _CPB, drain_body, 0)

  ld = start_load(0)
  for b in range(_NRB):
    ld.wait()
    if b >= 2:
      drain_block(b - 2)
    dst = rows[b % 2]

    def transpose_body(k, colvec, dst=dst):
      base = k * 8
      for u in range(8):
        col = plsc.load_gather(tbuf, [iota16, colvec + u])
        dst[base + u, :] = col
      return colvec + 8

    lax.fori_loop(
        0, _RB // 8, transpose_body, jnp.zeros((16,), jnp.int32)
    )
    if b + 1 < _NRB:
      ld = start_load(b + 1)
    fire_block(b)

  drain_block(_NRB - 2)
  drain_block(_NRB - 1)
  plsc.subcore_barrier()

  pltpu.sync_copy(
      acc_sh.at[pl.ds(s * _RPS, _RPS), :], rows_v0.at[pl.ds(0, _RPS), :]
  )
  pltpu.sync_copy(
      rows_v0.at[pl.ds(0, _RPS), :], out_hbm.at[c, pl.ds(s * _RPS, _RPS), :]
  )


def _segment_sum_sc(efeats, dst2d):
  mesh = plsc.VectorSubcoreMesh(
      core_axis_name="c", subcore_axis_name="s",
      num_cores=_NC, num_subcores=_NS,
  )
  return pl.kernel(
      _seg_sum_body,
      out_type=jax.ShapeDtypeStruct((_NC, _NPAD, D_E), jnp.float32),
      mesh=mesh,
      scratch_types=[
          pltpu.VMEM((_NCH, _CH), jnp.int32),
          pltpu.VMEM((16, _RB + 1), jnp.float32),
          pltpu.VMEM((_RB, D_E), jnp.float32),
          pltpu.VMEM((_RB, D_E), jnp.float32),
          pltpu.VMEM_SHARED((_NPAD, D_E), jnp.float32),
          pltpu.SemaphoreType.DMA,
          pltpu.SemaphoreType.DMA,
      ],
      compiler_params=pltpu.CompilerParams(
          use_tc_tiling_on_sc=False, needs_layout_passes=False
      ),
  )(efeats, dst2d)


_RBLK = 2000


def _mlp_body(nf_ref, p_ref, w1a_ref, w1b_ref, b1_ref, w2_ref, b2_ref, out_ref):
  hn = p_ref[0] + p_ref[1]
  x = jnp.dot(nf_ref[:], w1a_ref[:], preferred_element_type=jnp.float32)
  x = x + jnp.dot(hn, w1b_ref[:], preferred_element_type=jnp.float32)
  h1 = jnp.maximum(x + b1_ref[:], 0.0)
  y = jnp.dot(h1, w2_ref[:], preferred_element_type=jnp.float32) + b2_ref[:]
  out_ref[:] = jnp.maximum(y, 0.0)


def _mlp_tc(nfeats, partials, w1a, w1b, b1, w2, b2):
  grid = (N // _RBLK,)
  return pl.pallas_call(
      _mlp_body,
      grid=grid,
      in_specs=[
          pl.BlockSpec((_RBLK, D_IN), lambda i: (i, 0)),
          pl.BlockSpec((_NC, _RBLK, D_E), lambda i: (0, i, 0)),
          pl.BlockSpec((D_IN, D_OUT), lambda i: (0, 0)),
          pl.BlockSpec((D_E, D_OUT), lambda i: (0, 0)),
          pl.BlockSpec((1, D_OUT), lambda i: (0, 0)),
          pl.BlockSpec((D_OUT, D_OUT), lambda i: (0, 0)),
          pl.BlockSpec((1, D_OUT), lambda i: (0, 0)),
      ],
      out_specs=pl.BlockSpec((_RBLK, D_OUT), lambda i: (i, 0)),
      out_shape=jax.ShapeDtypeStruct((N, D_OUT), jnp.float32),
  )(nfeats, partials, w1a, w1b, b1, w2, b2)


def kernel(nfeats, efeats, edge_index, W1, b1, W2, b2):
  ei4d = edge_index.reshape(2, _NW, _NCH, _CH)
  partials = _segment_sum_sc(efeats.T, ei4d)
  w1a = W1[:D_IN]
  w1b = W1[D_IN:]
  return _mlp_tc(
      nfeats, partials, w1a, w1b,
      b1.reshape(1, D_OUT), W2, b2.reshape(1, D_OUT),
  )


# speedup vs baseline: 1.4436x; 1.4436x over previous
import functools

import jax
import jax.numpy as jnp
from jax import lax
from jax.experimental import pallas as pl
from jax.experimental.pallas import tpu as pltpu
from jax.experimental.pallas import tpu_sc as plsc

N, E, D_IN, D_E, D_OUT = 10000, 320000, 128, 16, 128

_NC, _NS = 2, 16
_NW = _NC * _NS
_EPW = E // _NW
_CH = 80
_NCH = _EPW // _CH
_RB = 2000
_NRB = _EPW // _RB
_CPB = _RB // _CH
_NPAD = 10240
_RPS = _NPAD // _NS


def _seg_sum_body(
    ef_t_hbm, dst_hbm, out_hbm,
    idx_v, tbuf, rows_v0, rows_v1, acc_sh, lsem, ssem,
):
  c = lax.axis_index("c")
  s = lax.axis_index("s")
  wid = s * _NC + c
  rows = (rows_v0, rows_v1)

  zeros16 = jnp.zeros((16,), jnp.float32)

  def zero_body(i, carry):
    rows_v0[i, :] = zeros16
    return carry

  lax.fori_loop(0, _RPS, zero_body, 0)
  pltpu.sync_copy(
      rows_v0.at[pl.ds(0, _RPS), :], acc_sh.at[pl.ds(s * _RPS, _RPS), :]
  )

  pltpu.sync_copy(dst_hbm.at[1, wid], idx_v)
  plsc.subcore_barrier()

  def start_load(b):
    return pltpu.async_copy(
        ef_t_hbm.at[:, pl.ds(wid * _EPW + b * _RB, _RB)],
        tbuf.at[:, pl.ds(0, _RB)],
        lsem,
    )

  iota16 = lax.iota(jnp.int32, 16)

  def fire_block(b):
    def fire_body(k, carry, b=b):
      pltpu.async_copy(
          rows[b % 2].at[pl.ds(k * _CH, _CH), :],
          acc_sh.at[idx_v.at[b * _CPB + k]],
          ssem,
          add=True,
      )
      return carry

    lax.fori_loop(0, _CPB, fire_body, 0)

  def drain_block(b):
    def drain_body(k, carry, b=b):
      pltpu.make_async_copy(
          rows[b % 2].at[pl.ds(k * _CH, _CH), :],
          acc_sh.at[idx_v.at[b * _CPB + k]],
          ssem,
      ).wait()
      return carry

    lax.fori_loop(0, _CPB, drain_body, 0)

  ld = start_load(0)
  for b in range(_NRB):
    ld.wait()
    if b >= 2:
      drain_block(b - 2)
    dst = rows[b % 2]

    def transpose_body(k, colvec, dst=dst):
      base = k * 8
      cols = [plsc.load_gather(tbuf, [iota16, colvec + u]) for u in range(8)]
      for u in range(8):
        dst[base + u, :] = cols[u]
      return colvec + 8

    lax.fori_loop(
        0, _RB // 8, transpose_body, jnp.zeros((16,), jnp.int32)
    )
    if b + 1 < _NRB:
      ld = start_load(b + 1)
    fire_block(b)

  drain_block(_NRB - 2)
  drain_block(_NRB - 1)
  plsc.subcore_barrier()

  pltpu.sync_copy(
      acc_sh.at[pl.ds(s * _RPS, _RPS), :], rows_v0.at[pl.ds(0, _RPS), :]
  )
  pltpu.sync_copy(
      rows_v0.at[pl.ds(0, _RPS), :], out_hbm.at[c, pl.ds(s * _RPS, _RPS), :]
  )


def _segment_sum_sc(efeats, dst2d):
  mesh = plsc.VectorSubcoreMesh(
      core_axis_name="c", subcore_axis_name="s",
      num_cores=_NC, num_subcores=_NS,
  )
  return pl.kernel(
      _seg_sum_body,
      out_type=jax.ShapeDtypeStruct((_NC, _NPAD, D_E), jnp.float32),
      mesh=mesh,
      scratch_types=[
          pltpu.VMEM((_NCH, _CH), jnp.int32),
          pltpu.VMEM((16, _RB + 1), jnp.float32),
          pltpu.VMEM((_RB, D_E), jnp.float32),
          pltpu.VMEM((_RB, D_E), jnp.float32),
          pltpu.VMEM_SHARED((_NPAD, D_E), jnp.float32),
          pltpu.SemaphoreType.DMA,
          pltpu.SemaphoreType.DMA,
      ],
      compiler_params=pltpu.CompilerParams(
          use_tc_tiling_on_sc=False, needs_layout_passes=False
      ),
  )(efeats, dst2d)


_RBLK = 2000


def _mlp_body(nf_ref, p_ref, w1a_ref, w1b_ref, b1_ref, w2_ref, b2_ref, out_ref):
  hn = p_ref[0] + p_ref[1]
  x = jnp.dot(nf_ref[:], w1a_ref[:], preferred_element_type=jnp.float32)
  x = x + jnp.dot(hn, w1b_ref[:], preferred_element_type=jnp.float32)
  h1 = jnp.maximum(x + b1_ref[:], 0.0)
  y = jnp.dot(h1, w2_ref[:], preferred_element_type=jnp.float32) + b2_ref[:]
  out_ref[:] = jnp.maximum(y, 0.0)


def _mlp_tc(nfeats, partials, w1a, w1b, b1, w2, b2):
  grid = (N // _RBLK,)
  return pl.pallas_call(
      _mlp_body,
      grid=grid,
      in_specs=[
          pl.BlockSpec((_RBLK, D_IN), lambda i: (i, 0)),
          pl.BlockSpec((_NC, _RBLK, D_E), lambda i: (0, i, 0)),
          pl.BlockSpec((D_IN, D_OUT), lambda i: (0, 0)),
          pl.BlockSpec((D_E, D_OUT), lambda i: (0, 0)),
          pl.BlockSpec((1, D_OUT), lambda i: (0, 0)),
          pl.BlockSpec((D_OUT, D_OUT), lambda i: (0, 0)),
          pl.BlockSpec((1, D_OUT), lambda i: (0, 0)),
      ],
      out_specs=pl.BlockSpec((_RBLK, D_OUT), lambda i: (i, 0)),
      out_shape=jax.ShapeDtypeStruct((N, D_OUT), jnp.float32),
  )(nfeats, partials, w1a, w1b, b1, w2, b2)


def kernel(nfeats, efeats, edge_index, W1, b1, W2, b2):
  ei4d = edge_index.reshape(2, _NW, _NCH, _CH)
  partials = _segment_sum_sc(efeats.T, ei4d)
  w1a = W1[:D_IN]
  w1b = W1[D_IN:]
  return _mlp_tc(
      nfeats, partials, w1a, w1b,
      b1.reshape(1, D_OUT), W2, b2.reshape(1, D_OUT),
  )


# speedup vs baseline: 1.6075x; 1.1135x over previous
import functools

import jax
import jax.numpy as jnp
from jax import lax
from jax.experimental import pallas as pl
from jax.experimental.pallas import tpu as pltpu
from jax.experimental.pallas import tpu_sc as plsc

N, E, D_IN, D_E, D_OUT = 10000, 320000, 128, 16, 128

_NC, _NS = 2, 16
_NW = _NC * _NS
_EPW = E // _NW
_CH = 80
_NCH = _EPW // _CH
_RB = 2000
_NRB = _EPW // _RB
_CPB = _RB // _CH
_NPAD = 10240
_RPS = _NPAD // _NS


def _seg_sum_body(
    ef_t_hbm, dst_hbm, out_hbm,
    idx_v, tbuf, rows_v0, rows_v1, acc_sh, lsem, ssem,
):
  c = lax.axis_index("c")
  s = lax.axis_index("s")
  wid = s * _NC + c
  rows = (rows_v0, rows_v1)

  zeros16 = jnp.zeros((16,), jnp.float32)

  def zero_body(i, carry):
    rows_v0[i, :] = zeros16
    return carry

  lax.fori_loop(0, _RPS, zero_body, 0)
  pltpu.sync_copy(
      rows_v0.at[pl.ds(0, _RPS), :], acc_sh.at[pl.ds(s * _RPS, _RPS), :]
  )

  pltpu.sync_copy(dst_hbm.at[1, wid], idx_v)
  plsc.subcore_barrier()

  def start_load(b):
    return pltpu.async_copy(
        ef_t_hbm.at[:, pl.ds(wid * _EPW + b * _RB, _RB)],
        tbuf.at[:, pl.ds(0, _RB)],
        lsem,
    )

  iota16 = lax.iota(jnp.int32, 16)

  def fire_block(b):
    def fire_body(k, carry, b=b):
      pltpu.async_copy(
          rows[b % 2].at[pl.ds(k * _CH, _CH), :],
          acc_sh.at[idx_v.at[b * _CPB + k]],
          ssem,
          add=True,
      )
      return carry

    lax.fori_loop(0, _CPB, fire_body, 0)

  def drain_block(b):
    def drain_body(k, carry, b=b):
      pltpu.make_async_copy(
          rows[b % 2].at[pl.ds(k * _CH, _CH), :],
          acc_sh.at[idx_v.at[b * _CPB + k]],
          ssem,
      ).wait()
      return carry

    lax.fori_loop(0, _CPB, drain_body, 0)

  ld = start_load(0)
  for b in range(_NRB):
    ld.wait()
    if b >= 2:
      drain_block(b - 2)
    dst = rows[b % 2]

    def transpose_body(k, colvec, dst=dst):
      base = k * 8
      cols = [plsc.load_gather(tbuf, [iota16, colvec + u]) for u in range(8)]
      for u in range(8):
        dst[base + u, :] = cols[u]
      return colvec + 8

    lax.fori_loop(
        0, _RB // 8, transpose_body, jnp.zeros((16,), jnp.int32)
    )
    if b + 1 < _NRB:
      ld = start_load(b + 1)
    fire_block(b)

  drain_block(_NRB - 2)
  drain_block(_NRB - 1)
  plsc.subcore_barrier()

  pltpu.sync_copy(
      acc_sh.at[pl.ds(s * _RPS, _RPS), :], rows_v0.at[pl.ds(0, _RPS), :]
  )
  pltpu.sync_copy(
      rows_v0.at[pl.ds(0, _RPS), :], out_hbm.at[c, pl.ds(s * _RPS, _RPS), :]
  )


def _segment_sum_sc(efeats, dst2d):
  mesh = plsc.VectorSubcoreMesh(
      core_axis_name="c", subcore_axis_name="s",
      num_cores=_NC, num_subcores=_NS,
  )
  return pl.kernel(
      _seg_sum_body,
      out_type=jax.ShapeDtypeStruct((_NC, _NPAD, D_E), jnp.float32),
      mesh=mesh,
      scratch_types=[
          pltpu.VMEM((_NCH, _CH), jnp.int32),
          pltpu.VMEM((16, _RB + 1), jnp.float32),
          pltpu.VMEM((_RB, D_E), jnp.float32),
          pltpu.VMEM((_RB, D_E), jnp.float32),
          pltpu.VMEM_SHARED((_NPAD, D_E), jnp.float32),
          pltpu.SemaphoreType.DMA,
          pltpu.SemaphoreType.DMA,
      ],
      compiler_params=pltpu.CompilerParams(
          use_tc_tiling_on_sc=False, needs_layout_passes=False
      ),
  )(efeats, dst2d)


_RBLK = 2048


def _mlp_body(nf_ref, pp_ref, w1a_ref, w1bd_ref, b1_ref, w2_ref, b2_ref, out_ref):
  hp = pp_ref[0] + pp_ref[1]
  xb = jnp.dot(hp, w1bd_ref[:], preferred_element_type=jnp.float32)
  x = xb.reshape(_RBLK, D_OUT)
  x = x + jnp.dot(nf_ref[:], w1a_ref[:], preferred_element_type=jnp.float32)
  h1 = jnp.maximum(x + b1_ref[:], 0.0)
  y = jnp.dot(h1, w2_ref[:], preferred_element_type=jnp.float32) + b2_ref[:]
  out_ref[:] = jnp.maximum(y, 0.0)


def _mlp_tc(nfeats, packed, w1a, w1bd, b1, w2, b2):
  grid = (pl.cdiv(N, _RBLK),)
  return pl.pallas_call(
      _mlp_body,
      grid=grid,
      in_specs=[
          pl.BlockSpec((_RBLK, D_IN), lambda i: (i, 0)),
          pl.BlockSpec((_NC, _RBLK // 8, 128), lambda i: (0, i, 0)),
          pl.BlockSpec((D_IN, D_OUT), lambda i: (0, 0)),
          pl.BlockSpec((128, 8 * D_OUT), lambda i: (0, 0)),
          pl.BlockSpec((1, D_OUT), lambda i: (0, 0)),
          pl.BlockSpec((D_OUT, D_OUT), lambda i: (0, 0)),
          pl.BlockSpec((1, D_OUT), lambda i: (0, 0)),
      ],
      out_specs=pl.BlockSpec((_RBLK, D_OUT), lambda i: (i, 0)),
      out_shape=jax.ShapeDtypeStruct((N, D_OUT), jnp.float32),
  )(nfeats, packed, w1a, w1bd, b1, w2, b2)


def kernel(nfeats, efeats, edge_index, W1, b1, W2, b2):
  ei4d = edge_index.reshape(2, _NW, _NCH, _CH)
  partials = _segment_sum_sc(efeats.T, ei4d)
  packed = partials.reshape(_NC, _NPAD // 8, 128)
  w1a = W1[:D_IN]
  w1b = W1[D_IN:]
  w1bd = (
      jnp.eye(8, dtype=jnp.float32)[:, None, :, None]
      * w1b[None, :, None, :]
  ).reshape(8 * D_E, 8 * D_OUT)
  return _mlp_tc(
      nfeats, packed, w1a, w1bd,
      b1.reshape(1, D_OUT), W2, b2.reshape(1, D_OUT),
  )


# speedup vs baseline: 1.9347x; 1.2036x over previous
import functools

import jax
import jax.numpy as jnp
from jax import lax
from jax.experimental import pallas as pl
from jax.experimental.pallas import tpu as pltpu
from jax.experimental.pallas import tpu_sc as plsc

N, E, D_IN, D_E, D_OUT = 10000, 320000, 128, 16, 128

_NC, _NS = 2, 16
_NW = _NC * _NS
_NCT = E // 128
_CTW = _NCT // _NW
_XTRA = _NCT - _CTW * _NW
_KCT = 6
_NBK = _CTW // _KCT
_RS = 129
_CS = 8 * _RS
_TS = 7 * _CS
_NPAD = 10240
_RPS = _NPAD // _NS


def _seg_sum_body(
    ef4_hbm, ei3_hbm, out_hbm,
    idx_v, tbuf0, tbuf1, rows_v0, rows_v1, acc_sh, lsem0, lsem1, ssem,
):
  c = lax.axis_index("c")
  s = lax.axis_index("s")
  wid = s * _NC + c
  rows = (rows_v0, rows_v1)
  tbufs = (tbuf0, tbuf1)
  lsems = (lsem0, lsem1)
  base_ct = wid * _CTW + jnp.minimum(wid, _XTRA)
  has_extra = wid < _XTRA

  zeros16 = jnp.zeros((16,), jnp.float32)

  def zero_body(i, carry):
    rows_v0[i, :] = zeros16
    return carry

  lax.fori_loop(0, _RPS, zero_body, 0)
  pltpu.sync_copy(
      rows_v0.at[pl.ds(0, _RPS), :], acc_sh.at[pl.ds(s * _RPS, _RPS), :]
  )

  pltpu.sync_copy(
      ei3_hbm.at[pl.ds(base_ct, _CTW), 1, :], idx_v.at[pl.ds(0, _CTW), :]
  )

  @pl.when(has_extra)
  def _():
    pltpu.sync_copy(
        ei3_hbm.at[pl.ds(base_ct + _CTW, 1), 1, :],
        idx_v.at[pl.ds(_CTW, 1), :],
    )

  plsc.subcore_barrier()

  def start_load(b):
    return pltpu.async_copy(
        ef4_hbm.at[:, pl.ds(base_ct + b * _KCT, _KCT)],
        tbufs[b % 2].at[:, pl.ds(0, _KCT), :, pl.ds(0, 128)],
        lsems[b % 2],
    )

  iota16 = lax.iota(jnp.int32, 16)
  tvec = iota16 // 8
  rvec = iota16 % 8

  def transpose_ct(tbuf, dstrows, ct):
    ctvec = jnp.full((16,), ct, jnp.int32)

    def tbody(k, colvec, tbuf=tbuf, dstrows=dstrows, ct=ct):
      base = ct * 128 + k * 8
      cols = [
          plsc.load_gather(tbuf, [tvec, ctvec, rvec, colvec + u])
          for u in range(8)
      ]
      for u in range(8):
        dstrows[base + u, :] = cols[u]
      return colvec + 8

    lax.fori_loop(0, 16, tbody, jnp.zeros((16,), jnp.int32))

  def fire_block(b):
    for ct in range(_KCT):
      pltpu.async_copy(
          rows[b % 2].at[pl.ds(ct * 128, 128), :],
          acc_sh.at[idx_v.at[b * _KCT + ct]],
          ssem,
          add=True,
      )

  def drain_block(b):
    for ct in range(_KCT):
      pltpu.make_async_copy(
          rows[b % 2].at[pl.ds(ct * 128, 128), :],
          acc_sh.at[idx_v.at[b * _KCT + ct]],
          ssem,
      ).wait()

  descs = {0: start_load(0), 1: start_load(1)}
  for b in range(_NBK):
    descs[b].wait()
    if b >= 2:
      drain_block(b - 2)
    for ct in range(_KCT):
      transpose_ct(tbufs[b % 2], rows[b % 2], ct)
    if b + 2 < _NBK:
      descs[b + 2] = start_load(b + 2)
    fire_block(b)

  drain_block(_NBK - 2)
  drain_block(_NBK - 1)

  @pl.when(has_extra)
  def _():
    pltpu.sync_copy(
        ef4_hbm.at[:, pl.ds(base_ct + _CTW, 1)],
        tbufs[0].at[:, pl.ds(0, 1), :, pl.ds(0, 128)],
    )
    transpose_ct(tbufs[0], rows_v1, 0)
    pltpu.sync_copy(
        rows_v1.at[pl.ds(0, 128), :],
        acc_sh.at[idx_v.at[_CTW]],
        add=True,
    )

  plsc.subcore_barrier()

  pltpu.sync_copy(
      acc_sh.at[pl.ds(s * _RPS, _RPS), :], rows_v0.at[pl.ds(0, _RPS), :]
  )
  pltpu.sync_copy(
      rows_v0.at[pl.ds(0, _RPS), :], out_hbm.at[c, pl.ds(s * _RPS, _RPS), :]
  )


def _segment_sum_sc(ef4, ei3):
  mesh = plsc.VectorSubcoreMesh(
      core_axis_name="c", subcore_axis_name="s",
      num_cores=_NC, num_subcores=_NS,
  )
  return pl.kernel(
      _seg_sum_body,
      out_type=jax.ShapeDtypeStruct((_NC, _NPAD, D_E), jnp.float32),
      mesh=mesh,
      scratch_types=[
          pltpu.VMEM((_CTW + 1, 128), jnp.int32),
          pltpu.VMEM((2, 7, 8, _RS), jnp.float32),
          pltpu.VMEM((2, 7, 8, _RS), jnp.float32),
          pltpu.VMEM((_KCT * 128, D_E), jnp.float32),
          pltpu.VMEM((_KCT * 128, D_E), jnp.float32),
          pltpu.VMEM_SHARED((_NPAD, D_E), jnp.float32),
          pltpu.SemaphoreType.DMA,
          pltpu.SemaphoreType.DMA,
          pltpu.SemaphoreType.DMA,
      ],
      compiler_params=pltpu.CompilerParams(
          use_tc_tiling_on_sc=False, needs_layout_passes=False
      ),
  )(ef4, ei3)


_RBLK = 2048


def _mlp_body(nf_ref, pp_ref, w1a_ref, w1bd_ref, b1_ref, w2_ref, b2_ref, out_ref):
  hp = pp_ref[0] + pp_ref[1]
  xb = jnp.dot(hp, w1bd_ref[:], preferred_element_type=jnp.float32)
  x = xb.reshape(_RBLK, D_OUT)
  x = x + jnp.dot(nf_ref[:], w1a_ref[:], preferred_element_type=jnp.float32)
  h1 = jnp.maximum(x + b1_ref[:], 0.0)
  y = jnp.dot(h1, w2_ref[:], preferred_element_type=jnp.float32) + b2_ref[:]
  out_ref[:] = jnp.maximum(y, 0.0)


def _mlp_tc(nfeats, packed, w1a, w1bd, b1, w2, b2):
  grid = (pl.cdiv(N, _RBLK),)
  return pl.pallas_call(
      _mlp_body,
      grid=grid,
      in_specs=[
          pl.BlockSpec((_RBLK, D_IN), lambda i: (i, 0)),
          pl.BlockSpec((_NC, _RBLK // 8, 128), lambda i: (0, i, 0)),
          pl.BlockSpec((D_IN, D_OUT), lambda i: (0, 0)),
          pl.BlockSpec((128, 8 * D_OUT), lambda i: (0, 0)),
          pl.BlockSpec((1, D_OUT), lambda i: (0, 0)),
          pl.BlockSpec((D_OUT, D_OUT), lambda i: (0, 0)),
          pl.BlockSpec((1, D_OUT), lambda i: (0, 0)),
      ],
      out_specs=pl.BlockSpec((_RBLK, D_OUT), lambda i: (i, 0)),
      out_shape=jax.ShapeDtypeStruct((N, D_OUT), jnp.float32),
  )(nfeats, packed, w1a, w1bd, b1, w2, b2)


def kernel(nfeats, efeats, edge_index, W1, b1, W2, b2):
  ef4 = efeats.T.reshape(2, 8, _NCT, 128).transpose(0, 2, 1, 3)
  ei3 = edge_index.reshape(2, _NCT, 128).transpose(1, 0, 2)
  partials = _segment_sum_sc(ef4, ei3)
  packed = partials.reshape(_NC, _NPAD // 8, 128)
  w1a = W1[:D_IN]
  w1b = W1[D_IN:]
  w1bd = (
      jnp.eye(8, dtype=jnp.float32)[:, None, :, None]
      * w1b[None, :, None, :]
  ).reshape(8 * D_E, 8 * D_OUT)
  return _mlp_tc(
      nfeats, packed, w1a, w1bd,
      b1.reshape(1, D_OUT), W2, b2.reshape(1, D_OUT),
  )
